# baseline (speedup 1.0000x reference)
import jax
import jax.numpy as jnp
from jax import lax
from jax.experimental import pallas as pl
from jax.experimental.pallas import tpu as pltpu

N = 1024
NB = 8
B = N // NB


def _lrelu(t):
    return jnp.where(t >= 0, t, 0.01 * t)


def _fused(ei_hbm, x_ref, W1_ref, b1_ref, W2_ref, b2_ref, W3_ref, b3_ref,
           W4_ref, b4_ref, W5_ref, b5_ref, recon_hbm, xr_ref, z_ref,
           ahat_ref, eibuf_ref, rbuf_ref, insem, outsem):
    def in_copy(b):
        return pltpu.make_async_copy(
            ei_hbm.at[pl.ds(b * B, B), :], eibuf_ref.at[b % 2], insem.at[b % 2])

    def out_copy(b):
        return pltpu.make_async_copy(
            rbuf_ref.at[b % 2], recon_hbm.at[pl.ds(b * B, B), :], outsem.at[b % 2])

    in_copy(0).start()
    deg = jnp.zeros((N,), jnp.float32)
    for b in range(NB):
        if b + 1 < NB:
            in_copy(b + 1).start()
        in_copy(b).wait()
        blk = (eibuf_ref[b % 2] != 0).astype(jnp.float32)
        rg = lax.broadcasted_iota(jnp.int32, (B, N), 0) + b * B
        cg = lax.broadcasted_iota(jnp.int32, (B, N), 1)
        ablk = jnp.where(rg == cg, 1.0, blk)
        ahat_ref[pl.ds(b * B, B), :] = ablk
        deg = deg + jnp.sum(ablk, axis=0)

    dinv = jnp.where(deg > 0, lax.rsqrt(deg), 0.0)
    dcol = dinv[:, None]
    ahat = ahat_ref[...]

    def conv(h, W_ref, b_ref):
        hw = jnp.dot(h, W_ref[...], preferred_element_type=jnp.float32)
        t = lax.dot_general(ahat, dcol * hw, (((0,), (0,)), ((), ())),
                            preferred_element_type=jnp.float32)
        return _lrelu(dcol * t + b_ref[...])

    h1 = conv(x_ref[...], W1_ref, b1_ref)
    z = conv(h1, W2_ref, b2_ref)
    re = conv(z, W3_ref, b3_ref)

    rdims = (((1,), (1,)), ((), ()))
    for b in range(NB):
        if b >= 2:
            out_copy(b - 2).wait()
        g = lax.dot_general(re[b * B:(b + 1) * B], re, rdims,
                            preferred_element_type=jnp.float32)
        rbuf_ref[b % 2] = jax.nn.sigmoid(g)
        out_copy(b).start()

    xh = conv(z, W4_ref, b4_ref)
    xr_ref[...] = conv(xh, W5_ref, b5_ref)
    z_ref[...] = z

    out_copy(NB - 2).wait()
    out_copy(NB - 1).wait()


def kernel(x, edge_index, W1, b1, W2, b2, W3, b3, W4, b4, W5, b5):
    ei = edge_index.astype(jnp.int32)
    biases = [b.reshape(1, -1) for b in (b1, b2, b3, b4, b5)]
    vmem = pl.BlockSpec(memory_space=pltpu.MemorySpace.VMEM)
    hbm = pl.BlockSpec(memory_space=pltpu.MemorySpace.HBM)
    out_shape = (
        jax.ShapeDtypeStruct((N, N), jnp.float32),
        jax.ShapeDtypeStruct((N, W5.shape[1]), jnp.float32),
        jax.ShapeDtypeStruct((N, W2.shape[1]), jnp.float32),
    )
    recon, xr, z = pl.pallas_call(
        _fused,
        out_shape=out_shape,
        in_specs=[hbm] + [vmem] * 11,
        out_specs=(hbm, vmem, vmem),
        scratch_shapes=[
            pltpu.VMEM((N, N), jnp.float32),
            pltpu.VMEM((2, B, N), jnp.int32),
            pltpu.VMEM((2, B, N), jnp.float32),
            pltpu.SemaphoreType.DMA((2,)),
            pltpu.SemaphoreType.DMA((2,)),
        ],
    )(ei, x, W1, biases[0], W2, biases[1], W3, biases[2],
      W4, biases[3], W5, biases[4])
    return (recon, xr, z)


# speedup vs baseline: 1.4305x; 1.4305x over previous
import jax
import jax.numpy as jnp
from jax import lax
from jax.experimental import pallas as pl
from jax.experimental.pallas import tpu as pltpu

N = 1024


def _lrelu(t):
    return jnp.where(t >= 0, t, 0.01 * t)


def _fused(ei_ref, x_ref, W1_hbm, b1_ref, W2_ref, b2_ref, W3_hbm, b3_ref,
           W4_hbm, b4_ref, W5_ref, b5_ref, recon_ref, xr_ref, z_ref,
           w1s, w3s, w4s, wsem):
    c1 = pltpu.make_async_copy(W1_hbm, w1s, wsem.at[0])
    c3 = pltpu.make_async_copy(W3_hbm, w3s, wsem.at[1])
    c4 = pltpu.make_async_copy(W4_hbm, w4s, wsem.at[2])
    c1.start(); c3.start(); c4.start()

    adj = (ei_ref[...] != 0).astype(jnp.float32)
    r = lax.broadcasted_iota(jnp.int32, (N, N), 0)
    c = lax.broadcasted_iota(jnp.int32, (N, N), 1)
    ahat = jnp.where(r == c, 1.0, adj)
    deg = jnp.sum(ahat, axis=0)
    dinv = jnp.where(deg > 0, lax.rsqrt(deg), 0.0)
    dcol = dinv[:, None]

    c1.wait(); c3.wait(); c4.wait()

    def agg(hw, b):
        t = lax.dot_general(ahat, dcol * hw, (((0,), (0,)), ((), ())),
                            preferred_element_type=jnp.float32)
        return dcol * t + b

    def mm(h, W):
        return jnp.dot(h, W, preferred_element_type=jnp.float32)

    h1 = _lrelu(agg(mm(x_ref[...], w1s[...]), b1_ref[...]))
    z = _lrelu(agg(mm(h1, W2_ref[...]), b2_ref[...]))
    w34 = jnp.concatenate([w3s[...], w4s[...]], axis=1)
    b34 = jnp.concatenate([b3_ref[...], b4_ref[...]], axis=1)
    t34 = agg(mm(z, w34), b34)
    re = _lrelu(t34[:, :64])
    xh = _lrelu(t34[:, 64:])
    recon_ref[...] = jax.nn.sigmoid(
        lax.dot_general(re, re, (((1,), (1,)), ((), ())),
                        preferred_element_type=jnp.float32))
    xr_ref[...] = _lrelu(agg(mm(xh, W5_ref[...]), b5_ref[...]))
    z_ref[...] = z


def kernel(x, edge_index, W1, b1, W2, b2, W3, b3, W4, b4, W5, b5):
    ei = edge_index.astype(jnp.int32)
    vmem = pl.BlockSpec(memory_space=pltpu.MemorySpace.VMEM)
    hbm = pl.BlockSpec(memory_space=pltpu.MemorySpace.HBM)
    out_shape = (
        jax.ShapeDtypeStruct((N, N), jnp.float32),
        jax.ShapeDtypeStruct((N, W5.shape[1]), jnp.float32),
        jax.ShapeDtypeStruct((N, W2.shape[1]), jnp.float32),
    )
    recon, xr, z = pl.pallas_call(
        _fused,
        out_shape=out_shape,
        in_specs=[vmem, vmem, hbm, vmem, vmem, vmem, hbm, vmem, hbm, vmem,
                  vmem, vmem],
        out_specs=(vmem, vmem, vmem),
        scratch_shapes=[
            pltpu.VMEM((128, 64), jnp.float32),
            pltpu.VMEM((128, 64), jnp.float32),
            pltpu.VMEM((128, 64), jnp.float32),
            pltpu.SemaphoreType.DMA((3,)),
        ],
    )(ei, x, W1, b1.reshape(1, -1), W2, b2.reshape(1, -1), W3,
      b3.reshape(1, -1), W4, b4.reshape(1, -1), W5, b5.reshape(1, -1))
    return (recon, xr, z)


# speedup vs baseline: 1.8437x; 1.2888x over previous
import jax
import jax.numpy as jnp
from jax import lax
from jax.experimental import pallas as pl

N = 1024


def _lrelu(t):
    return jnp.where(t >= 0, t, 0.01 * t)


def _fused(ei_ref, x_ref, wp_ref, b1_ref, W2_ref, b2_ref, b3_ref,
           b4_ref, W5_ref, b5_ref, recon_ref, xr_ref, z_ref):
    adj = (ei_ref[...] != 0).astype(jnp.float32)
    r = lax.broadcasted_iota(jnp.int32, (N, N), 0)
    c = lax.broadcasted_iota(jnp.int32, (N, N), 1)
    ahat = jnp.where(r == c, 1.0, adj)
    deg = jnp.sum(ahat, axis=0)
    dinv = jnp.where(deg > 0, lax.rsqrt(deg), 0.0)
    dcol = dinv[:, None]

    w1 = wp_ref[0:64, :].reshape(128, 64)
    w3 = wp_ref[64:128, :].reshape(128, 64)
    w4 = wp_ref[128:192, :].reshape(128, 64)
    w34 = jnp.concatenate([w3, w4], axis=1)
    b34 = jnp.concatenate([b3_ref[...], b4_ref[...]], axis=1)

    def agg(hw, b):
        t = lax.dot_general(ahat, dcol * hw, (((0,), (0,)), ((), ())),
                            preferred_element_type=jnp.float32)
        return dcol * t + b

    def mm(h, W):
        return jnp.dot(h, W, preferred_element_type=jnp.float32)

    h1 = _lrelu(agg(mm(x_ref[...], w1), b1_ref[...]))
    z = _lrelu(agg(mm(h1, W2_ref[...]), b2_ref[...]))
    t34 = agg(mm(z, w34), b34)
    re = _lrelu(t34[:, :64])
    xh = _lrelu(t34[:, 64:])
    recon_ref[...] = jax.nn.sigmoid(
        lax.dot_general(re, re, (((1,), (1,)), ((), ())),
                        preferred_element_type=jnp.float32))
    xr_ref[...] = _lrelu(agg(mm(xh, W5_ref[...]), b5_ref[...]))
    z_ref[...] = z


def kernel(x, edge_index, W1, b1, W2, b2, W3, b3, W4, b4, W5, b5):
    ei = edge_index.astype(jnp.int32)
    wpack = jnp.concatenate([W1, W3, W4], axis=0).reshape(192, 128)
    out_shape = (
        jax.ShapeDtypeStruct((N, N), jnp.float32),
        jax.ShapeDtypeStruct((N, W5.shape[1]), jnp.float32),
        jax.ShapeDtypeStruct((N, W2.shape[1]), jnp.float32),
    )
    recon, xr, z = pl.pallas_call(
        _fused,
        out_shape=out_shape,
    )(ei, x, wpack, b1.reshape(1, -1), W2, b2.reshape(1, -1),
      b3.reshape(1, -1), b4.reshape(1, -1), W5, b5.reshape(1, -1))
    return (recon, xr, z)
